# baseline (speedup 1.0000x reference)
import functools

import jax
import jax.numpy as jnp
from jax import lax
from jax.experimental import pallas as pl
from jax.experimental.pallas import tpu as pltpu
from jax.experimental.pallas import tpu_sc as plsc

INV_SQRT_2 = 0.5 ** 0.5

_NS = 16
_BE = 128
_W = 128
_K = 3
_CH = 8


def _sc_aggregate(N, SLAB, NB, xr, packed):
    mesh = plsc.VectorSubcoreMesh(
        core_axis_name="c", subcore_axis_name="s", num_cores=1)
    rows_per_tile = SLAB // _NS
    zrows = 32
    nvec = _CH * _BE // 16

    @functools.partial(
        pl.kernel,
        out_type=jax.ShapeDtypeStruct((_K * SLAB, _W), jnp.float32),
        mesh=mesh,
        scratch_types=[
            pltpu.VMEM((_CH, _BE), jnp.int32),
            pltpu.VMEM((_CH, _BE), jnp.int32),
            pltpu.VMEM((_BE, _W), jnp.float32),
            pltpu.VMEM((_BE, _W), jnp.float32),
            pltpu.VMEM((zrows, _W), jnp.float32),
            pltpu.VMEM_SHARED((SLAB, _W), jnp.float32),
            pltpu.SemaphoreType.DMA,
            pltpu.SemaphoreType.DMA,
        ],
    )
    def agg_kernel(xr_hbm, packed_hbm, out_hbm,
                   gi, di, buf0, buf1, zbuf, slab, sem0, sem1):
        s = lax.axis_index("s")
        nz = _W // 16
        ncol = _BE // 16

        def zero_zbuf(i, _):
            zbuf[i // nz, pl.ds((i % nz) * 16, 16)] = jnp.zeros((16,), jnp.float32)
            return 0
        lax.fori_loop(0, zrows * nz, zero_zbuf, 0)

        def zero_slab(k, _):
            pltpu.sync_copy(zbuf, slab.at[pl.ds(s * rows_per_tile + k * zrows, zrows)])
            return 0

        def accumulate(p):
            def chunk_step(c, _):
                pltpu.sync_copy(
                    packed_hbm.at[pl.ds(s * NB + c * _CH, _CH)], gi)

                def unpack(i, _):
                    r, cc = i // ncol, (i % ncol) * 16
                    v = gi[r, pl.ds(cc, 16)]
                    di[r, pl.ds(cc, 16)] = v & 0xFFFF
                    gi[r, pl.ds(cc, 16)] = lax.shift_right_logical(v, 16) + p
                    return 0
                lax.fori_loop(0, nvec, unpack, 0)

                pltpu.async_copy(xr_hbm.at[gi.at[0]], buf0, sem0)

                def step(jj, _):
                    j0 = jj * 2
                    pltpu.async_copy(xr_hbm.at[gi.at[j0 + 1]], buf1, sem1)
                    pltpu.make_async_copy(xr_hbm.at[gi.at[j0]], buf0, sem0).wait()
                    pltpu.sync_copy(buf0, slab.at[di.at[j0]], add=True)

                    @pl.when(jj < _CH // 2 - 1)
                    def _():
                        pltpu.async_copy(xr_hbm.at[gi.at[j0 + 2]], buf0, sem0)

                    pltpu.make_async_copy(xr_hbm.at[gi.at[j0 + 1]], buf1, sem1).wait()
                    pltpu.sync_copy(buf1, slab.at[di.at[j0 + 1]], add=True)
                    return 0
                lax.fori_loop(0, _CH // 2, step, 0)
                return 0
            lax.fori_loop(0, NB // _CH, chunk_step, 0)

        def out_copy(p):
            def chunk(k, _):
                r0 = s * rows_per_tile + k * _BE
                pltpu.sync_copy(slab.at[pl.ds(r0, _BE)], buf0)
                pltpu.sync_copy(buf0, out_hbm.at[pl.ds(p * SLAB + r0, _BE)])
                return 0
            lax.fori_loop(0, rows_per_tile // _BE, chunk, 0)

        for p in range(_K):
            lax.fori_loop(0, rows_per_tile // zrows, zero_slab, 0)
            plsc.subcore_barrier()
            accumulate(p)
            plsc.subcore_barrier()
            out_copy(p)
            if p + 1 < _K:
                plsc.subcore_barrier()

    return agg_kernel(xr, packed)


def _tc_stage(xr):
    M = xr.shape[0]
    BM = 2000

    def body(a_ref, o_ref):
        o_ref[...] = a_ref[...]

    return pl.pallas_call(
        body,
        grid=(M // BM,),
        in_specs=[pl.BlockSpec((BM, _W), lambda i: (i, 0))],
        out_specs=pl.BlockSpec((BM, _W), lambda i: (i, 0)),
        out_shape=jax.ShapeDtypeStruct((M, _W), jnp.float32),
    )(xr)


def _tc_combine(N, F, x2d, agg, norm2d, Mn, Me):
    BN = 400
    grid = (N // BN,)

    def body(x_ref, a0_ref, a1_ref, a2_ref, nrm_ref,
             mn_ref, me0_ref, me1_ref, me2_ref, o_ref):
        f32 = jnp.float32
        y = jnp.dot(x_ref[...], mn_ref[...], preferred_element_type=f32)
        a = jnp.dot(a0_ref[...], me0_ref[...], preferred_element_type=f32)
        a += jnp.dot(a1_ref[...], me1_ref[...], preferred_element_type=f32)
        a += jnp.dot(a2_ref[...], me2_ref[...], preferred_element_type=f32)
        o_ref[...] = (y + nrm_ref[...] * a) * INV_SQRT_2

    part = pl.BlockSpec((BN, _W), lambda i: (i, 0))
    wspec = pl.BlockSpec((_W, F), lambda i: (0, 0))
    return pl.pallas_call(
        body,
        grid=grid,
        in_specs=[
            pl.BlockSpec((BN, F), lambda i: (i, 0)),
            part, part, part,
            pl.BlockSpec((BN, 1), lambda i: (i, 0)),
            pl.BlockSpec((F, F), lambda i: (0, 0)),
            wspec, wspec, wspec,
        ],
        out_specs=pl.BlockSpec((BN, F), lambda i: (i, 0)),
        out_shape=jax.ShapeDtypeStruct((N, F), jnp.float32),
    )(x2d, agg[0], agg[1], agg[2], norm2d,
      Mn, Me[:_W], Me[_W:2 * _W], Me[2 * _W:])


def kernel(x, src, dst, norm_coeff, W_node, W_edge):
    B, N, DI, K = x.shape
    E = src.shape[0]
    F = DI * K

    unit = _NS * _BE * 8
    E_pad = -(-E // unit) * unit
    pad = E_pad - E
    SLAB = ((N + _NS * _BE - 1) // (_NS * _BE)) * (_NS * _BE)

    src_p = jnp.concatenate([src, jnp.zeros((pad,), jnp.int32)])
    dst_p = jnp.concatenate([dst, jnp.full((pad,), N, jnp.int32)])
    packed = ((src_p * 3) << 16 | dst_p).reshape(-1, _BE)
    NB = packed.shape[0] // _NS

    xr = _tc_stage(x.reshape(N * K, _W))
    agg = _sc_aggregate(N, SLAB, NB, xr, packed).reshape(K, SLAB, _W)

    eye = jnp.eye(K, dtype=jnp.float32)
    Mn = jnp.kron(W_node.T, eye)
    Me = jnp.kron(W_edge.T, eye)
    x2d = x.reshape(N, F)
    out2d = _tc_combine(N, F, x2d, agg, norm_coeff.reshape(N, 1), Mn, Me)
    return out2d.reshape(B, N, DI, K)


# speedup vs baseline: 14.9738x; 14.9738x over previous
import functools

import jax
import jax.numpy as jnp
from jax import lax
from jax.experimental import pallas as pl
from jax.experimental.pallas import tpu as pltpu
from jax.experimental.pallas import tpu_sc as plsc

INV_SQRT_2 = 0.5 ** 0.5

_NC = 2
_NS = 16
_BE = 128
_W = 128
_K = 3
_CH = 16


def _sc_aggregate(N, SLAB, NB, xr, packed):
    mesh = plsc.VectorSubcoreMesh(
        core_axis_name="c", subcore_axis_name="s", num_cores=_NC)
    rows_per_tile = SLAB // _NS
    zrows = 32
    nvec = _CH * _BE // 16

    @functools.partial(
        pl.kernel,
        out_type=jax.ShapeDtypeStruct((_NC * _K * SLAB, _W), jnp.float32),
        mesh=mesh,
        scratch_types=[
            pltpu.VMEM((_CH, _BE), jnp.int32),
            pltpu.VMEM((_CH, _BE), jnp.int32),
            pltpu.VMEM((_BE, _W), jnp.float32),
            pltpu.VMEM((_BE, _W), jnp.float32),
            pltpu.VMEM((zrows, _W), jnp.float32),
            pltpu.VMEM_SHARED((SLAB, _W), jnp.float32),
            pltpu.SemaphoreType.DMA,
            pltpu.SemaphoreType.DMA,
        ],
    )
    def agg_kernel(xr_hbm, packed_hbm, out_hbm,
                   gi, di, buf0, buf1, zbuf, slab, sem0, sem1):
        cid = lax.axis_index("c")
        s = lax.axis_index("s")
        w = cid * _NS + s
        nz = _W // 16
        ncol = _BE // 16

        def zero_zbuf(i, _):
            zbuf[i // nz, pl.ds((i % nz) * 16, 16)] = jnp.zeros((16,), jnp.float32)
            return 0
        lax.fori_loop(0, zrows * nz, zero_zbuf, 0)

        def zero_slab(k, _):
            pltpu.sync_copy(zbuf, slab.at[pl.ds(s * rows_per_tile + k * zrows, zrows)])
            return 0
        lax.fori_loop(0, rows_per_tile // zrows, zero_slab, 0)

        def accumulate(p):
            def chunk_step(c, _):
                pltpu.sync_copy(
                    packed_hbm.at[pl.ds(w * NB + c * _CH, _CH)], gi)

                def unpack(i, _):
                    r, cc = i // ncol, (i % ncol) * 16
                    v = gi[r, pl.ds(cc, 16)]
                    di[r, pl.ds(cc, 16)] = v & 0xFFFF
                    gi[r, pl.ds(cc, 16)] = (
                        lax.shift_right_logical(v, 16) + p * N)
                    return 0
                lax.fori_loop(0, nvec, unpack, 0)

                pltpu.async_copy(xr_hbm.at[gi.at[0]], buf0, sem0)

                def step(jj, _):
                    j0 = jj * 2
                    pltpu.async_copy(xr_hbm.at[gi.at[j0 + 1]], buf1, sem1)
                    pltpu.make_async_copy(xr_hbm.at[gi.at[j0]], buf0, sem0).wait()
                    pltpu.sync_copy(buf0, slab.at[di.at[j0]], add=True)

                    @pl.when(jj < _CH // 2 - 1)
                    def _():
                        pltpu.async_copy(xr_hbm.at[gi.at[j0 + 2]], buf0, sem0)

                    pltpu.make_async_copy(xr_hbm.at[gi.at[j0 + 1]], buf1, sem1).wait()
                    pltpu.sync_copy(buf1, slab.at[di.at[j0 + 1]], add=True)
                    return 0
                lax.fori_loop(0, _CH // 2, step, 0)
                return 0
            lax.fori_loop(0, NB // _CH, chunk_step, 0)

        def out_copy(p, rezero):
            def chunk(k, _):
                r0 = s * rows_per_tile + k * _BE
                pltpu.sync_copy(slab.at[pl.ds(r0, _BE)], buf0)
                pltpu.sync_copy(
                    buf0,
                    out_hbm.at[pl.ds((cid * _K + p) * SLAB + r0, _BE)])
                if rezero:
                    def z(q, _):
                        pltpu.sync_copy(
                            zbuf, slab.at[pl.ds(r0 + q * zrows, zrows)])
                        return 0
                    lax.fori_loop(0, _BE // zrows, z, 0)
                return 0
            lax.fori_loop(0, rows_per_tile // _BE, chunk, 0)

        plsc.subcore_barrier()
        for p in range(_K):
            accumulate(p)
            plsc.subcore_barrier()
            out_copy(p, p + 1 < _K)
            if p + 1 < _K:
                plsc.subcore_barrier()

    return agg_kernel(xr, packed)


def _tc_combine(N, xt3, agg3, norm2d, WnT, WeT):
    BN = 1000

    def body(x_ref, a0_ref, a1_ref, nrm_ref, wn_ref, we_ref, o_ref):
        f32 = jnp.float32
        y = jnp.dot(x_ref[0], wn_ref[...], preferred_element_type=f32)
        a = jnp.dot(a0_ref[0] + a1_ref[0], we_ref[...],
                    preferred_element_type=f32)
        o_ref[0] = (y + nrm_ref[...] * a) * INV_SQRT_2

    wspec = pl.BlockSpec((_W, _W), lambda k, j: (0, 0))
    return pl.pallas_call(
        body,
        grid=(_K, N // BN),
        in_specs=[
            pl.BlockSpec((1, BN, _W), lambda k, j: (k, j, 0)),
            pl.BlockSpec((1, BN, _W), lambda k, j: (k, j, 0)),
            pl.BlockSpec((1, BN, _W), lambda k, j: (_K + k, j, 0)),
            pl.BlockSpec((BN, 1), lambda k, j: (j, 0)),
            wspec, wspec,
        ],
        out_specs=pl.BlockSpec((1, BN, _W), lambda k, j: (k, j, 0)),
        out_shape=jax.ShapeDtypeStruct((_K, N, _W), jnp.float32),
    )(xt3, agg3, agg3, norm2d, WnT, WeT)


def kernel(x, src, dst, norm_coeff, W_node, W_edge):
    B, N, DI, K = x.shape
    E = src.shape[0]

    unit = _NC * _NS * _BE * _CH
    E_pad = -(-E // unit) * unit
    pad = E_pad - E
    SLAB = ((N + 1 + _NS * _BE - 1) // (_NS * _BE)) * (_NS * _BE)

    spread = jnp.arange(pad, dtype=jnp.int32)
    src_p = jnp.concatenate([src, spread % N])
    dst_p = jnp.concatenate([dst, N + spread % (SLAB - N)])
    packed = ((src_p << 16) | dst_p).reshape(-1, _BE)
    NB = packed.shape[0] // (_NC * _NS)

    xt = jnp.transpose(x, (0, 3, 1, 2)).reshape(K * N, DI)
    agg = _sc_aggregate(N, SLAB, NB, xt, packed)

    y = _tc_combine(N, xt.reshape(K, N, DI), agg.reshape(_NC * K, SLAB, _W),
                    norm_coeff.reshape(N, 1), W_node.T, W_edge.T)
    return jnp.transpose(y, (1, 2, 0)).reshape(B, N, DI, K)


# speedup vs baseline: 15.3030x; 1.0220x over previous
import functools

import jax
import jax.numpy as jnp
from jax import lax
from jax.experimental import pallas as pl
from jax.experimental.pallas import tpu as pltpu
from jax.experimental.pallas import tpu_sc as plsc

INV_SQRT_2 = 0.5 ** 0.5

_NC = 2
_NS = 16
_BE = 128
_W = 128
_K = 3
_CH = 16


def _sc_aggregate(N, SLAB, NB, xr, packed):
    mesh = plsc.VectorSubcoreMesh(
        core_axis_name="c", subcore_axis_name="s", num_cores=_NC)
    rows_per_tile = SLAB // _NS
    zrows = 32
    nvec = _CH * _BE // 16

    @functools.partial(
        pl.kernel,
        out_type=jax.ShapeDtypeStruct((_NC * _K * SLAB, _W), jnp.float32),
        mesh=mesh,
        scratch_types=[
            pltpu.VMEM((_CH, _BE), jnp.int32),
            pltpu.VMEM((_CH, _BE), jnp.int32),
            pltpu.VMEM((_BE, _W), jnp.float32),
            pltpu.VMEM((_BE, _W), jnp.float32),
            pltpu.VMEM((zrows, _W), jnp.float32),
            pltpu.VMEM_SHARED((SLAB, _W), jnp.float32),
            pltpu.SemaphoreType.DMA,
            pltpu.SemaphoreType.DMA,
        ],
    )
    def agg_kernel(xr_hbm, packed_hbm, out_hbm,
                   gi, di, buf0, buf1, zbuf, slab, sem0, sem1):
        cid = lax.axis_index("c")
        s = lax.axis_index("s")
        w = cid * _NS + s
        nz = _W // 16
        ncol = _BE // 16

        def zero_zbuf(i, _):
            zbuf[i // nz, pl.ds((i % nz) * 16, 16)] = jnp.zeros((16,), jnp.float32)
            return 0
        lax.fori_loop(0, zrows * nz, zero_zbuf, 0)

        def zero_slab(k, _):
            pltpu.sync_copy(zbuf, slab.at[pl.ds(s * rows_per_tile + k * zrows, zrows)])
            return 0
        lax.fori_loop(0, rows_per_tile // zrows, zero_slab, 0)

        def accumulate(p):
            def chunk_step(c, _):
                pltpu.sync_copy(
                    packed_hbm.at[pl.ds(w * NB + c * _CH, _CH)], gi)

                def unpack(i, _):
                    r, cc = i // ncol, (i % ncol) * 16
                    v = gi[r, pl.ds(cc, 16)]
                    di[r, pl.ds(cc, 16)] = v & 0xFFFF
                    gi[r, pl.ds(cc, 16)] = (
                        lax.shift_right_logical(v, 16) + p * N)
                    return 0
                lax.fori_loop(0, nvec, unpack, 0)

                pltpu.async_copy(xr_hbm.at[gi.at[0]], buf0, sem0)

                def step(jj, _):
                    j0 = jj * 2
                    pltpu.async_copy(xr_hbm.at[gi.at[j0 + 1]], buf1, sem1)
                    pltpu.make_async_copy(xr_hbm.at[gi.at[j0]], buf0, sem0).wait()
                    pltpu.sync_copy(buf0, slab.at[di.at[j0]], add=True)

                    @pl.when(jj < _CH // 2 - 1)
                    def _():
                        pltpu.async_copy(xr_hbm.at[gi.at[j0 + 2]], buf0, sem0)

                    pltpu.make_async_copy(xr_hbm.at[gi.at[j0 + 1]], buf1, sem1).wait()
                    pltpu.sync_copy(buf1, slab.at[di.at[j0 + 1]], add=True)
                    return 0
                lax.fori_loop(0, _CH // 2, step, 0)
                return 0
            lax.fori_loop(0, NB // _CH, chunk_step, 0)

        def out_copy(p, rezero):
            def chunk(k, _):
                r0 = s * rows_per_tile + k * _BE
                pltpu.sync_copy(slab.at[pl.ds(r0, _BE)], buf0)
                pltpu.sync_copy(
                    buf0,
                    out_hbm.at[pl.ds((cid * _K + p) * SLAB + r0, _BE)])
                if rezero:
                    def z(q, _):
                        pltpu.sync_copy(
                            zbuf, slab.at[pl.ds(r0 + q * zrows, zrows)])
                        return 0
                    lax.fori_loop(0, _BE // zrows, z, 0)
                return 0
            lax.fori_loop(0, rows_per_tile // _BE, chunk, 0)

        plsc.subcore_barrier()
        for p in range(_K):
            accumulate(p)
            plsc.subcore_barrier()
            out_copy(p, p + 1 < _K)
            if p + 1 < _K:
                plsc.subcore_barrier()

    return agg_kernel(xr, packed)


def _tc_combine(N, xt3, agg3, norm2d, WnT, WeT):
    BN = 2000

    def body(x_ref, a0_ref, a1_ref, nrm_ref, wn_ref, we_ref, o_ref):
        f32 = jnp.float32
        y = jnp.dot(x_ref[0], wn_ref[...], preferred_element_type=f32)
        a = jnp.dot(a0_ref[0] + a1_ref[0], we_ref[...],
                    preferred_element_type=f32)
        o_ref[0] = (y + nrm_ref[...] * a) * INV_SQRT_2

    wspec = pl.BlockSpec((_W, _W), lambda k, j: (0, 0))
    return pl.pallas_call(
        body,
        grid=(_K, N // BN),
        in_specs=[
            pl.BlockSpec((1, BN, _W), lambda k, j: (k, j, 0)),
            pl.BlockSpec((1, BN, _W), lambda k, j: (k, j, 0)),
            pl.BlockSpec((1, BN, _W), lambda k, j: (_K + k, j, 0)),
            pl.BlockSpec((BN, 1), lambda k, j: (j, 0)),
            wspec, wspec,
        ],
        out_specs=pl.BlockSpec((1, BN, _W), lambda k, j: (k, j, 0)),
        out_shape=jax.ShapeDtypeStruct((_K, N, _W), jnp.float32),
    )(xt3, agg3, agg3, norm2d, WnT, WeT)


def kernel(x, src, dst, norm_coeff, W_node, W_edge):
    B, N, DI, K = x.shape
    E = src.shape[0]

    unit = _NC * _NS * _BE * _CH
    E_pad = -(-E // unit) * unit
    pad = E_pad - E
    SLAB = ((N + 1 + _NS * _BE - 1) // (_NS * _BE)) * (_NS * _BE)

    spread = jnp.arange(pad, dtype=jnp.int32)
    src_p = jnp.concatenate([src, spread % N])
    dst_p = jnp.concatenate([dst, N + spread % (SLAB - N)])
    packed = ((src_p << 16) | dst_p).reshape(-1, _BE)
    NB = packed.shape[0] // (_NC * _NS)

    xt = jnp.transpose(x, (0, 3, 1, 2)).reshape(K * N, DI)
    agg = _sc_aggregate(N, SLAB, NB, xt, packed)

    y = _tc_combine(N, xt.reshape(K, N, DI), agg.reshape(_NC * K, SLAB, _W),
                    norm_coeff.reshape(N, 1), W_node.T, W_edge.T)
    return jnp.transpose(y, (1, 2, 0)).reshape(B, N, DI, K)
